# Initial kernel scaffold; baseline (speedup 1.0000x reference)
import functools

import jax
import jax.numpy as jnp
from jax import lax
from jax.experimental import pallas as pl
from jax.experimental.pallas import tpu as pltpu
from jax.experimental.pallas import tpu_sc as plsc

N = 10000
NP = 10240
E = 320000
H = 128
WROW = 144
TRASH = 10000
KP = 1664
BM = 128
NT = 16
NCH = 158
EP = NT * NCH * 128
ROWS_PER_TILE = NP // NT


def _leaky(v):
    return jnp.where(v > 0, v, 0.01 * v)


def _front_body(x_ref, wbig_ref, bbig_ref, win_ref, bin_ref, pa_ref,
                wroot_ref, broot_ref, wr0_ref, wr1_ref, root_ref, yy_ref):
    x = x_ref[...]
    h1 = jnp.dot(x, wbig_ref[...], preferred_element_type=jnp.float32)
    h1 = _leaky(h1 + bbig_ref[...])
    h = jnp.dot(h1, win_ref[...], preferred_element_type=jnp.float32)
    h = h + bin_ref[...]
    h = jnp.where(h > 0, h, pa_ref[...] * h)
    root_ref[...] = (
        jnp.dot(h, wroot_ref[...], preferred_element_type=jnp.float32)
        + broot_ref[...])
    ones = jnp.ones((BM, 1), jnp.float32)
    zer = jnp.zeros((BM, WROW - H - 1), jnp.float32)
    yy_ref[0] = jnp.concatenate(
        [jnp.dot(h, wr0_ref[...], preferred_element_type=jnp.float32), ones, zer], axis=1)
    yy_ref[1] = jnp.concatenate(
        [jnp.dot(h, wr1_ref[...], preferred_element_type=jnp.float32), ones, zer], axis=1)


def _combine(root, s):
    a0 = s[0, :, :H] / jnp.maximum(s[0, :, H:H + 1], 1.0)
    a1 = s[1, :, :H] / jnp.maximum(s[1, :, H:H + 1], 1.0)
    return root + a0 + a1


def _mid_body(root_ref, s_ref, wroot_ref, broot_ref, wr0_ref, wr1_ref,
              root_out_ref, yy_ref):
    h = _combine(root_ref[...], s_ref[...])
    root_out_ref[...] = (
        jnp.dot(h, wroot_ref[...], preferred_element_type=jnp.float32)
        + broot_ref[...])
    ones = jnp.ones((BM, 1), jnp.float32)
    zer = jnp.zeros((BM, WROW - H - 1), jnp.float32)
    yy_ref[0] = jnp.concatenate(
        [jnp.dot(h, wr0_ref[...], preferred_element_type=jnp.float32), ones, zer], axis=1)
    yy_ref[1] = jnp.concatenate(
        [jnp.dot(h, wr1_ref[...], preferred_element_type=jnp.float32), ones, zer], axis=1)


def _final_body(root_ref, s_ref, wcls_ref, bcls_ref, out_ref):
    h = _combine(root_ref[...], s_ref[...])
    out_ref[...] = (
        jnp.dot(h, wcls_ref[...], preferred_element_type=jnp.float32)
        + bcls_ref[...])


def _prep_body(src_ref, dst_ref, et_ref, g_ref, s_ref):
    s = src_ref[...]
    d = dst_ref[...]
    t = et_ref[...]
    g_ref[0] = s
    g_ref[1] = s + NP
    s_ref[0] = jnp.where(t == 0, d, TRASH)
    s_ref[1] = jnp.where(t == 1, d, TRASH)


def _FULL(shape):
    return pl.BlockSpec(shape, lambda i: tuple(0 for _ in shape))


def _ROWB():
    return pl.BlockSpec((BM, H), lambda i: (i, 0))


def _YYB():
    return pl.BlockSpec((2, BM, WROW), lambda i: (0, i, 0))


def _front(xp, wbig, bbig, win, b_in, pa, wroot, broot, wr0, wr1):
    return pl.pallas_call(
        _front_body,
        grid=(NP // BM,),
        in_specs=[
            pl.BlockSpec((BM, KP), lambda i: (i, 0)),
            _FULL((KP, H)), _FULL((1, H)), _FULL((H, H)), _FULL((1, H)),
            _FULL((1, H)), _FULL((H, H)), _FULL((1, H)), _FULL((H, H)),
            _FULL((H, H)),
        ],
        out_specs=[_ROWB(), _YYB()],
        out_shape=[
            jax.ShapeDtypeStruct((NP, H), jnp.float32),
            jax.ShapeDtypeStruct((2, NP, WROW), jnp.float32),
        ],
    )(xp, wbig, bbig, win, b_in, pa, wroot, broot, wr0, wr1)


def _mid(root, s, wroot, broot, wr0, wr1):
    return pl.pallas_call(
        _mid_body,
        grid=(NP // BM,),
        in_specs=[
            _ROWB(), _YYB(),
            _FULL((H, H)), _FULL((1, H)), _FULL((H, H)), _FULL((H, H)),
        ],
        out_specs=[_ROWB(), _YYB()],
        out_shape=[
            jax.ShapeDtypeStruct((NP, H), jnp.float32),
            jax.ShapeDtypeStruct((2, NP, WROW), jnp.float32),
        ],
    )(root, s, wroot, broot, wr0, wr1)


def _final(root, s, wcls, bcls):
    return pl.pallas_call(
        _final_body,
        grid=(NP // BM,),
        in_specs=[_ROWB(), _YYB(), _FULL((H, H)), _FULL((1, H))],
        out_specs=_ROWB(),
        out_shape=jax.ShapeDtypeStruct((NP, H), jnp.float32),
    )(root, s, wcls, bcls)


def _prep(src2d, dst2d, et2d):
    nrow = EP // 128
    return pl.pallas_call(
        _prep_body,
        grid=(NCH,),
        in_specs=[pl.BlockSpec((nrow // NCH, 128), lambda i: (i, 0))] * 3,
        out_specs=[pl.BlockSpec((2, nrow // NCH, 128), lambda i: (0, i, 0))] * 2,
        out_shape=[jax.ShapeDtypeStruct((2, nrow, 128), jnp.int32)] * 2,
    )(src2d, dst2d, et2d)


def _sc_agg_body(yy_hbm, gidx_hbm, sidx_hbm, out_hbm,
                 gidx_v, sidx_v, buf0, buf1, acc, sem):
    c = lax.axis_index("c")
    s = lax.axis_index("s")
    row0 = s * ROWS_PER_TILE

    pltpu.sync_copy(gidx_hbm.at[c, s], gidx_v)
    pltpu.sync_copy(sidx_hbm.at[c, s], sidx_v)

    def _zrow(i, carry):
        for j in range(WROW // 16):
            buf0[i, pl.ds(j * 16, 16)] = jnp.zeros((16,), jnp.float32)
        return carry
    lax.fori_loop(0, 128, _zrow, 0)

    def _zcopy(k, carry):
        pltpu.sync_copy(buf0, acc.at[pl.ds(row0 + k * 128, 128)])
        return carry
    lax.fori_loop(0, ROWS_PER_TILE // 128, _zcopy, 0)
    plsc.subcore_barrier()

    pltpu.async_copy(yy_hbm.at[gidx_v.at[0]], buf0, sem)

    def _mbody(g, carry):
        for b in range(2):
            bufa = buf0 if b == 0 else buf1
            bufb = buf1 if b == 0 else buf0
            j = g * 2 + b
            pltpu.make_async_copy(yy_hbm.at[gidx_v.at[j]], bufa, sem).wait()
            jn = jnp.minimum(j + 1, NCH - 1)
            pltpu.async_copy(yy_hbm.at[gidx_v.at[jn]], bufb, sem)
            pltpu.sync_copy(bufa, acc.at[sidx_v.at[j]], add=True)
        return carry
    lax.fori_loop(0, NCH // 2, _mbody, 0)
    pltpu.make_async_copy(yy_hbm.at[gidx_v.at[0]], buf0, sem).wait()
    plsc.subcore_barrier()

    def _obody(k, carry):
        pltpu.sync_copy(acc.at[pl.ds(row0 + k * 128, 128)], buf1)
        pltpu.sync_copy(buf1, out_hbm.at[c, pl.ds(row0 + k * 128, 128)])
        return carry
    lax.fori_loop(0, ROWS_PER_TILE // 128, _obody, 0)


_sc_agg = functools.partial(
    pl.kernel,
    out_type=jax.ShapeDtypeStruct((2, NP, WROW), jnp.float32),
    mesh=plsc.VectorSubcoreMesh(
        core_axis_name="c", subcore_axis_name="s",
        num_cores=2, num_subcores=NT),
    scratch_types=[
        pltpu.VMEM((NCH, 128), jnp.int32),
        pltpu.VMEM((NCH, 128), jnp.int32),
        pltpu.VMEM((128, WROW), jnp.float32),
        pltpu.VMEM((128, WROW), jnp.float32),
        pltpu.VMEM_SHARED((NP, WROW), jnp.float32),
        pltpu.SemaphoreType.DMA,
    ],
)(_sc_agg_body)


def kernel(x, edge_index, edge_type, W_des, b_des, W_tweet, b_tweet,
           W_num, b_num, W_cat, b_cat, W_in, b_in, prelu_a,
           W_rel1, W_root1, b1, W_rel2, W_root2, b2, W_cls, b_cls):
    f32 = jnp.float32
    D_NUM, D_TWEET, D_CAT = 5, 768, 3

    xp = jnp.pad(x, ((0, NP - N), (0, KP - x.shape[1])))
    wbig = jnp.zeros((KP, H), f32)
    wbig = wbig.at[0:D_NUM, 64:96].set(W_num)
    wbig = wbig.at[D_NUM:D_NUM + D_TWEET, 32:64].set(W_tweet)
    wbig = wbig.at[D_NUM + D_TWEET:D_NUM + D_TWEET + D_CAT, 96:128].set(W_cat)
    wbig = wbig.at[D_NUM + D_TWEET + D_CAT:D_NUM + 2 * D_TWEET + D_CAT,
                   0:32].set(W_des)
    bbig = jnp.concatenate([b_des, b_tweet, b_num, b_cat])[None, :]

    src = jnp.pad(edge_index[0].astype(jnp.int32), (0, EP - E))
    dst = jnp.pad(edge_index[1].astype(jnp.int32), (0, EP - E))
    et = jnp.pad(edge_type.astype(jnp.int32), (0, EP - E),
                 constant_values=2)
    nrow = EP // 128
    gidx, sidx = _prep(src.reshape(nrow, 128), dst.reshape(nrow, 128),
                       et.reshape(nrow, 128))
    gidx = gidx.reshape(2, NT, NCH, 128)
    sidx = sidx.reshape(2, NT, NCH, 128)

    root1, yy1 = _front(xp, wbig, bbig, W_in, b_in[None, :],
                        prelu_a[None, :], W_root1, b1[None, :],
                        W_rel1[0], W_rel1[1])
    s1 = _sc_agg(yy1.reshape(2 * NP, WROW), gidx, sidx)
    root2, yy2 = _mid(root1, s1, W_root2, b2[None, :], W_rel2[0], W_rel2[1])
    s2 = _sc_agg(yy2.reshape(2 * NP, WROW), gidx, sidx)
    out = _final(root2, s2, W_cls, b_cls[None, :])
    return out[:N]


# speedup vs baseline: 2.7966x; 2.7966x over previous
import functools

import jax
import jax.numpy as jnp
from jax import lax
from jax.experimental import pallas as pl
from jax.experimental.pallas import tpu as pltpu
from jax.experimental.pallas import tpu_sc as plsc

N = 10000
NP = 10240
E = 320000
H = 128
TRASH = 10000
KP = 1664
BM = 128
NT = 16
CPB = 16
NB = 10
NCH = CPB * NB
EP = NT * NCH * 128
ROWS_PER_TILE = NP // NT


def _leaky(v):
    return jnp.where(v > 0, v, 0.01 * v)


def _front_body(x_ref, wbig_ref, bbig_ref, win_ref, bin_ref, pa_ref,
                wroot_ref, broot_ref, wr0_ref, wr1_ref, root_ref, yy_ref):
    x = x_ref[...]
    h1 = jnp.dot(x, wbig_ref[...], preferred_element_type=jnp.float32)
    h1 = _leaky(h1 + bbig_ref[...])
    h = jnp.dot(h1, win_ref[...], preferred_element_type=jnp.float32)
    h = h + bin_ref[...]
    h = jnp.where(h > 0, h, pa_ref[...] * h)
    root_ref[...] = (
        jnp.dot(h, wroot_ref[...], preferred_element_type=jnp.float32)
        + broot_ref[...])
    yy_ref[0] = jnp.dot(h, wr0_ref[...], preferred_element_type=jnp.float32)
    yy_ref[1] = jnp.dot(h, wr1_ref[...], preferred_element_type=jnp.float32)


def _combine(root, s, cnt):
    c0 = jnp.maximum(cnt[0, :, 0:1], 1.0)
    c1 = jnp.maximum(cnt[1, :, 0:1], 1.0)
    return root + s[0] / c0 + s[1] / c1


def _mid_body(root_ref, s_ref, cnt_ref, wroot_ref, broot_ref, wr0_ref,
              wr1_ref, root_out_ref, yy_ref):
    h = _combine(root_ref[...], s_ref[...], cnt_ref[...])
    root_out_ref[...] = (
        jnp.dot(h, wroot_ref[...], preferred_element_type=jnp.float32)
        + broot_ref[...])
    yy_ref[0] = jnp.dot(h, wr0_ref[...], preferred_element_type=jnp.float32)
    yy_ref[1] = jnp.dot(h, wr1_ref[...], preferred_element_type=jnp.float32)


def _final_body(root_ref, s_ref, cnt_ref, wcls_ref, bcls_ref, out_ref):
    h = _combine(root_ref[...], s_ref[...], cnt_ref[...])
    out_ref[...] = (
        jnp.dot(h, wcls_ref[...], preferred_element_type=jnp.float32)
        + bcls_ref[...])


def _prep_body(src_ref, dst_ref, et_ref, g_ref, s_ref):
    s = src_ref[...]
    d = dst_ref[...]
    t = et_ref[...]
    g_ref[0] = s
    g_ref[1] = s + NP
    s_ref[0] = jnp.where(t == 0, d, TRASH)
    s_ref[1] = jnp.where(t == 1, d, TRASH)


def _FULL(shape):
    return pl.BlockSpec(shape, lambda i: tuple(0 for _ in shape))


def _ROWB():
    return pl.BlockSpec((BM, H), lambda i: (i, 0))


def _YYB():
    return pl.BlockSpec((2, BM, H), lambda i: (0, i, 0))


def _front(xp, wbig, bbig, win, b_in, pa, wroot, broot, wr0, wr1):
    return pl.pallas_call(
        _front_body,
        grid=(NP // BM,),
        in_specs=[
            pl.BlockSpec((BM, KP), lambda i: (i, 0)),
            _FULL((KP, H)), _FULL((1, H)), _FULL((H, H)), _FULL((1, H)),
            _FULL((1, H)), _FULL((H, H)), _FULL((1, H)), _FULL((H, H)),
            _FULL((H, H)),
        ],
        out_specs=[_ROWB(), _YYB()],
        out_shape=[
            jax.ShapeDtypeStruct((NP, H), jnp.float32),
            jax.ShapeDtypeStruct((2, NP, H), jnp.float32),
        ],
    )(xp, wbig, bbig, win, b_in, pa, wroot, broot, wr0, wr1)


def _mid(root, s, cnt, wroot, broot, wr0, wr1):
    return pl.pallas_call(
        _mid_body,
        grid=(NP // BM,),
        in_specs=[
            _ROWB(), _YYB(), _YYB(),
            _FULL((H, H)), _FULL((1, H)), _FULL((H, H)), _FULL((H, H)),
        ],
        out_specs=[_ROWB(), _YYB()],
        out_shape=[
            jax.ShapeDtypeStruct((NP, H), jnp.float32),
            jax.ShapeDtypeStruct((2, NP, H), jnp.float32),
        ],
    )(root, s, cnt, wroot, broot, wr0, wr1)


def _final(root, s, cnt, wcls, bcls):
    return pl.pallas_call(
        _final_body,
        grid=(NP // BM,),
        in_specs=[_ROWB(), _YYB(), _YYB(), _FULL((H, H)), _FULL((1, H))],
        out_specs=_ROWB(),
        out_shape=jax.ShapeDtypeStruct((NP, H), jnp.float32),
    )(root, s, cnt, wcls, bcls)


def _prep(src2d, dst2d, et2d):
    nrow = EP // 128
    return pl.pallas_call(
        _prep_body,
        grid=(NCH,),
        in_specs=[pl.BlockSpec((nrow // NCH, 128), lambda i: (i, 0))] * 3,
        out_specs=[pl.BlockSpec((2, nrow // NCH, 128), lambda i: (0, i, 0))] * 2,
        out_shape=[jax.ShapeDtypeStruct((2, nrow, 128), jnp.int32)] * 2,
    )(src2d, dst2d, et2d)


def _zero_buf(buf):
    def _zrow(i, carry):
        for j in range(H // 16):
            buf[i, pl.ds(j * 16, 16)] = jnp.zeros((16,), jnp.float32)
        return carry
    lax.fori_loop(0, 128, _zrow, 0)


def _zero_acc(buf, acc, row0):
    def _zcopy(k, carry):
        pltpu.sync_copy(buf, acc.at[pl.ds(row0 + k * 128, 128)])
        return carry
    lax.fori_loop(0, ROWS_PER_TILE // 128, _zcopy, 0)


def _write_out(buf, acc, out_hbm, c, row0):
    def _obody(k, carry):
        pltpu.sync_copy(acc.at[pl.ds(row0 + k * 128, 128)], buf)
        pltpu.sync_copy(buf, out_hbm.at[c, pl.ds(row0 + k * 128, 128)])
        return carry
    lax.fori_loop(0, ROWS_PER_TILE // 128, _obody, 0)


def _sc_agg_body(yy_hbm, gidx_hbm, sidx_hbm, out_hbm,
                 gidx_v, sidx_v, buf0, buf1, acc, sem):
    c = lax.axis_index("c")
    s = lax.axis_index("s")
    row0 = s * ROWS_PER_TILE

    _zero_buf(buf0)
    _zero_acc(buf0, acc, row0)
    plsc.subcore_barrier()

    def _block(nb, carry):
        pltpu.sync_copy(gidx_hbm.at[c, s, nb], gidx_v)
        pltpu.sync_copy(sidx_hbm.at[c, s, nb], sidx_v)
        pltpu.async_copy(yy_hbm.at[gidx_v.at[0]], buf0, sem)

        def _mbody(g, carry2):
            for b in range(2):
                bufa = buf0 if b == 0 else buf1
                bufb = buf1 if b == 0 else buf0
                j = g * 2 + b
                pltpu.make_async_copy(yy_hbm.at[gidx_v.at[j]], bufa,
                                      sem).wait()
                jn = jnp.minimum(j + 1, CPB - 1)
                pltpu.async_copy(yy_hbm.at[gidx_v.at[jn]], bufb, sem)
                pltpu.sync_copy(bufa, acc.at[sidx_v.at[j]], add=True)
            return carry2
        lax.fori_loop(0, CPB // 2, _mbody, 0)
        pltpu.make_async_copy(yy_hbm.at[gidx_v.at[0]], buf0, sem).wait()
        return carry
    lax.fori_loop(0, NB, _block, 0)
    plsc.subcore_barrier()

    _write_out(buf1, acc, out_hbm, c, row0)


def _sc_cnt_body(sidx_hbm, out_hbm, sidx_v, buf0, ones_v, acc):
    c = lax.axis_index("c")
    s = lax.axis_index("s")
    row0 = s * ROWS_PER_TILE

    _zero_buf(buf0)
    e0 = jnp.where(lax.broadcasted_iota(jnp.int32, (16,), 0) == 0, 1.0, 0.0)

    def _orow(i, carry):
        ones_v[i, pl.ds(0, 16)] = e0
        for j in range(1, H // 16):
            ones_v[i, pl.ds(j * 16, 16)] = jnp.zeros((16,), jnp.float32)
        return carry
    lax.fori_loop(0, 128, _orow, 0)

    _zero_acc(buf0, acc, row0)
    plsc.subcore_barrier()

    def _block(nb, carry):
        pltpu.sync_copy(sidx_hbm.at[c, s, nb], sidx_v)

        def _mbody(j, carry2):
            pltpu.sync_copy(ones_v, acc.at[sidx_v.at[j]], add=True)
            return carry2
        lax.fori_loop(0, CPB, _mbody, 0)
        return carry
    lax.fori_loop(0, NB, _block, 0)
    plsc.subcore_barrier()

    _write_out(buf0, acc, out_hbm, c, row0)


def _sc_mesh():
    return plsc.VectorSubcoreMesh(
        core_axis_name="c", subcore_axis_name="s",
        num_cores=2, num_subcores=NT)


@functools.cache
def _make_sc_agg():
    return pl.kernel(
        _sc_agg_body,
        out_type=jax.ShapeDtypeStruct((2, NP, H), jnp.float32),
        mesh=_sc_mesh(),
        scratch_types=[
            pltpu.VMEM((CPB, 128), jnp.int32),
            pltpu.VMEM((CPB, 128), jnp.int32),
            pltpu.VMEM((128, H), jnp.float32),
            pltpu.VMEM((128, H), jnp.float32),
            pltpu.VMEM_SHARED((NP, H), jnp.float32),
            pltpu.SemaphoreType.DMA,
        ],
    )


@functools.cache
def _make_sc_cnt():
    return pl.kernel(
        _sc_cnt_body,
        out_type=jax.ShapeDtypeStruct((2, NP, H), jnp.float32),
        mesh=_sc_mesh(),
        scratch_types=[
            pltpu.VMEM((CPB, 128), jnp.int32),
            pltpu.VMEM((128, H), jnp.float32),
            pltpu.VMEM((128, H), jnp.float32),
            pltpu.VMEM_SHARED((NP, H), jnp.float32),
        ],
    )


def kernel(x, edge_index, edge_type, W_des, b_des, W_tweet, b_tweet,
           W_num, b_num, W_cat, b_cat, W_in, b_in, prelu_a,
           W_rel1, W_root1, b1, W_rel2, W_root2, b2, W_cls, b_cls):
    f32 = jnp.float32
    D_NUM, D_TWEET, D_CAT = 5, 768, 3

    xp = jnp.pad(x, ((0, NP - N), (0, KP - x.shape[1])))
    wbig = jnp.zeros((KP, H), f32)
    wbig = wbig.at[0:D_NUM, 64:96].set(W_num)
    wbig = wbig.at[D_NUM:D_NUM + D_TWEET, 32:64].set(W_tweet)
    wbig = wbig.at[D_NUM + D_TWEET:D_NUM + D_TWEET + D_CAT, 96:128].set(W_cat)
    wbig = wbig.at[D_NUM + D_TWEET + D_CAT:D_NUM + 2 * D_TWEET + D_CAT,
                   0:32].set(W_des)
    bbig = jnp.concatenate([b_des, b_tweet, b_num, b_cat])[None, :]

    src = jnp.pad(edge_index[0].astype(jnp.int32), (0, EP - E))
    dst = jnp.pad(edge_index[1].astype(jnp.int32), (0, EP - E))
    et = jnp.pad(edge_type.astype(jnp.int32), (0, EP - E),
                 constant_values=2)
    nrow = EP // 128
    gidx, sidx = _prep(src.reshape(nrow, 128), dst.reshape(nrow, 128),
                       et.reshape(nrow, 128))
    gidx = gidx.reshape(2, NT, NB, CPB, 128)
    sidx = sidx.reshape(2, NT, NB, CPB, 128)

    cnts = _make_sc_cnt()(sidx)
    root1, yy1 = _front(xp, wbig, bbig, W_in, b_in[None, :],
                        prelu_a[None, :], W_root1, b1[None, :],
                        W_rel1[0], W_rel1[1])
    s1 = _make_sc_agg()(yy1.reshape(2 * NP, H), gidx, sidx)
    root2, yy2 = _mid(root1, s1, cnts, W_root2, b2[None, :],
                      W_rel2[0], W_rel2[1])
    s2 = _make_sc_agg()(yy2.reshape(2 * NP, H), gidx, sidx)
    out = _final(root2, s2, cnts, W_cls, b_cls[None, :])
    return out[:N]
